# baseline (speedup 1.0000x reference)
import functools

import jax
import jax.numpy as jnp
from jax import lax
from jax.experimental import pallas as pl
from jax.experimental.pallas import tpu as pltpu


def _round_up(x, m):
    return (x + m - 1) // m * m


def _fused_kernel(x_ref, w1_ref, b1_ref, w2_ref, b2_ref, o_ref, *, n_classes):
    xb = x_ref[...]

    h = lax.dot_general(
        xb, w1_ref[...],
        dimension_numbers=(((1,), (1,)), ((), ())),
        preferred_element_type=jnp.float32,
    )
    h = jnp.maximum(h + b1_ref[...], 0.0)

    logits = lax.dot_general(
        h, w2_ref[...],
        dimension_numbers=(((1,), (1,)), ((), ())),
        preferred_element_type=jnp.float32,
    ) + b2_ref[...]

    m = jnp.max(logits, axis=1, keepdims=True)
    e = jnp.exp(logits - m)
    denom = jnp.sum(e, axis=1, keepdims=True)
    del n_classes
    o_ref[...] = e * (1.0 / denom)


def kernel(x, w1, b1, w2, b2):
    B, in_f = x.shape
    hid = w1.shape[0]
    C = w2.shape[0]
    Cp = _round_up(C, 128)

    w1b = w1
    w2b = jnp.pad(w2, ((0, Cp - C), (0, 0)))
    b1r = b1.reshape(1, hid).astype(jnp.float32)
    b2r = jnp.pad(b2.astype(jnp.float32), (0, Cp - C),
                  constant_values=-1e30).reshape(1, Cp)

    tb = min(2048, B)
    grid = (pl.cdiv(B, tb),)

    x_tile = _round_up(tb, 8) * _round_up(in_f, 128) * 4
    o_tile = _round_up(tb, 8) * Cp * 4
    vmem_limit_bytes = int(min(
        max(2 * (x_tile + o_tile) + (6 << 20), 32 << 20), 100 << 20))

    out = pl.pallas_call(
        functools.partial(_fused_kernel, n_classes=C),
        out_shape=jax.ShapeDtypeStruct((B, Cp), jnp.float32),
        grid=grid,
        in_specs=[
            pl.BlockSpec((tb, in_f), lambda i: (i, 0)),
            pl.BlockSpec((hid, in_f), lambda i: (0, 0)),
            pl.BlockSpec((1, hid), lambda i: (0, 0)),
            pl.BlockSpec((Cp, hid), lambda i: (0, 0)),
            pl.BlockSpec((1, Cp), lambda i: (0, 0)),
        ],
        out_specs=pl.BlockSpec((tb, Cp), lambda i: (i, 0)),
        compiler_params=pltpu.CompilerParams(
            dimension_semantics=("parallel",),
            vmem_limit_bytes=vmem_limit_bytes,
        ),
        cost_estimate=pl.CostEstimate(
            flops=2 * B * (in_f * hid + hid * C),
            transcendentals=B * Cp,
            bytes_accessed=4 * (B * in_f + B * C) + 2 * (hid * in_f + Cp * hid),
        ),
    )(x, w1b, b1r, w2b, b2r)
    return lax.slice(out, (0, 0), (B, C))


# speedup vs baseline: 1.2144x; 1.2144x over previous
import jax
import jax.numpy as jnp
from jax import lax
from jax.experimental import pallas as pl
from jax.experimental.pallas import tpu as pltpu


def _round_up(x, m):
    return (x + m - 1) // m * m


def _fused_kernel(x_ref, w1_ref, b1_ref, w2_ref, b2_ref, o_ref):
    h = lax.dot_general(
        x_ref[...], w1_ref[...],
        dimension_numbers=(((1,), (1,)), ((), ())),
        preferred_element_type=jnp.float32,
    )
    h = jnp.maximum(h + b1_ref[...], 0.0)

    logits = lax.dot_general(
        h, w2_ref[...],
        dimension_numbers=(((1,), (1,)), ((), ())),
        preferred_element_type=jnp.float32,
    ) + b2_ref[...]

    m = jnp.max(logits, axis=1, keepdims=True)
    e = jnp.exp(logits - m)
    denom = jnp.sum(e, axis=1, keepdims=True)
    o_ref[...] = e * (1.0 / denom)


def kernel(x, w1, b1, w2, b2):
    B, in_f = x.shape
    hid = w1.shape[0]
    C = w2.shape[0]

    b1r = b1.reshape(1, hid)
    b2r = b2.reshape(1, C)

    tb = min(1024, B)
    grid = (pl.cdiv(B, tb),)

    x_tile = _round_up(tb, 8) * _round_up(in_f, 128) * 4
    o_tile = _round_up(tb, 8) * _round_up(C, 128) * 4
    vmem_limit_bytes = int(min(
        max(2 * (x_tile + o_tile) + (6 << 20), 32 << 20), 100 << 20))

    return pl.pallas_call(
        _fused_kernel,
        out_shape=jax.ShapeDtypeStruct((B, C), jnp.float32),
        grid=grid,
        in_specs=[
            pl.BlockSpec((tb, in_f), lambda i: (i, 0)),
            pl.BlockSpec((hid, in_f), lambda i: (0, 0)),
            pl.BlockSpec((1, hid), lambda i: (0, 0)),
            pl.BlockSpec((C, hid), lambda i: (0, 0)),
            pl.BlockSpec((1, C), lambda i: (0, 0)),
        ],
        out_specs=pl.BlockSpec((tb, C), lambda i: (i, 0)),
        compiler_params=pltpu.CompilerParams(
            dimension_semantics=("parallel",),
            vmem_limit_bytes=vmem_limit_bytes,
        ),
        cost_estimate=pl.CostEstimate(
            flops=2 * B * (in_f * hid + hid * C),
            transcendentals=B * C,
            bytes_accessed=4 * (B * in_f + B * C + hid * in_f + C * hid),
        ),
    )(x, w1, b1r, w2, b2r)


# speedup vs baseline: 1.4380x; 1.1842x over previous
import jax
import jax.numpy as jnp
from jax import lax
from jax.experimental import pallas as pl
from jax.experimental.pallas import tpu as pltpu


def _round_up(x, m):
    return (x + m - 1) // m * m


def _fused_kernel(x_ref, w1_ref, b1_ref, w2_ref, b2_ref, o_ref):
    h = lax.dot_general(
        w1_ref[...], x_ref[...],
        dimension_numbers=(((1,), (1,)), ((), ())),
        preferred_element_type=jnp.float32,
    )
    b1c = b1_ref[...].reshape(w1_ref.shape[0], 1)
    h = jnp.maximum(h + b1c, 0.0)

    b2c = b2_ref[...].reshape(w2_ref.shape[0], 1)
    logits = jnp.dot(w2_ref[...], h, preferred_element_type=jnp.float32) + b2c

    m = jnp.max(logits, axis=0, keepdims=True)
    e = jnp.exp(logits - m)
    denom = jnp.sum(e, axis=0, keepdims=True)
    o_ref[...] = e * (1.0 / denom)


def kernel(x, w1, b1, w2, b2):
    B, in_f = x.shape
    hid = w1.shape[0]
    C = w2.shape[0]

    tb = min(1024, B)
    grid = (pl.cdiv(B, tb),)

    x_tile = _round_up(tb, 8) * _round_up(in_f, 128) * 4
    o_tile = _round_up(C, 8) * _round_up(tb, 128) * 4
    vmem_limit_bytes = int(min(
        max(2 * (x_tile + o_tile) + (6 << 20), 32 << 20), 100 << 20))

    out_cb = pl.pallas_call(
        _fused_kernel,
        out_shape=jax.ShapeDtypeStruct((C, B), jnp.float32),
        grid=grid,
        in_specs=[
            pl.BlockSpec((tb, in_f), lambda i: (i, 0)),
            pl.BlockSpec((hid, in_f), lambda i: (0, 0)),
            pl.BlockSpec((1, hid), lambda i: (0, 0)),
            pl.BlockSpec((C, hid), lambda i: (0, 0)),
            pl.BlockSpec((1, C), lambda i: (0, 0)),
        ],
        out_specs=pl.BlockSpec((C, tb), lambda i: (0, i)),
        compiler_params=pltpu.CompilerParams(
            dimension_semantics=("parallel",),
            vmem_limit_bytes=vmem_limit_bytes,
        ),
        cost_estimate=pl.CostEstimate(
            flops=2 * B * (in_f * hid + hid * C),
            transcendentals=B * C,
            bytes_accessed=4 * (B * in_f + B * C + hid * in_f + C * hid),
        ),
    )(x, w1, b1.reshape(1, hid), w2, b2.reshape(1, C))

    return out_cb.T


# speedup vs baseline: 1.4400x; 1.0014x over previous
import jax
import jax.numpy as jnp
from jax import lax
from jax.experimental import pallas as pl
from jax.experimental.pallas import tpu as pltpu


def _round_up(x, m):
    return (x + m - 1) // m * m


def _fused_kernel(x_ref, w1_ref, b1_ref, w2_ref, b2_ref, o_ref):
    h = lax.dot_general(
        w1_ref[...], x_ref[...],
        dimension_numbers=(((1,), (1,)), ((), ())),
        preferred_element_type=jnp.float32,
    )
    b1c = b1_ref[...].reshape(b1_ref.shape[-1], 1)
    h = jnp.maximum(h + b1c, 0.0)

    b2c = b2_ref[...].reshape(b2_ref.shape[-1], 1)
    logits = jnp.dot(w2_ref[...], h, preferred_element_type=jnp.float32) + b2c

    m = jnp.max(logits, axis=0, keepdims=True)
    e = jnp.exp(logits - m)
    denom = jnp.sum(e, axis=0, keepdims=True)
    o_ref[...] = e * (1.0 / denom)


def kernel(x, w1, b1, w2, b2):
    B, in_f = x.shape
    hid = w1.shape[0]
    C = w2.shape[0]

    tb = min(1024, B)
    grid = (pl.cdiv(B, tb),)

    x_tile = _round_up(tb, 8) * _round_up(in_f, 128) * 4
    o_tile = _round_up(C, 8) * _round_up(tb, 128) * 4
    vmem_limit_bytes = int(min(
        max(2 * (x_tile + o_tile) + (6 << 20), 32 << 20), 100 << 20))

    out_cb = pl.pallas_call(
        _fused_kernel,
        out_shape=jax.ShapeDtypeStruct((C, B), jnp.float32),
        grid=grid,
        in_specs=[
            pl.BlockSpec((tb, in_f), lambda i: (i, 0)),
            pl.BlockSpec((hid, in_f), lambda i: (0, 0)),
            pl.BlockSpec((hid,), lambda i: (0,)),
            pl.BlockSpec((C, hid), lambda i: (0, 0)),
            pl.BlockSpec((C,), lambda i: (0,)),
        ],
        out_specs=pl.BlockSpec((C, tb), lambda i: (0, i)),
        compiler_params=pltpu.CompilerParams(
            dimension_semantics=("parallel",),
            vmem_limit_bytes=vmem_limit_bytes,
        ),
        cost_estimate=pl.CostEstimate(
            flops=2 * B * (in_f * hid + hid * C),
            transcendentals=B * C,
            bytes_accessed=4 * (B * in_f + B * C + hid * in_f + C * hid),
        ),
    )(x, w1, b1, w2, b2)

    return out_cb.T
